# baseline (speedup 1.0000x reference)
import math

import jax
import jax.numpy as jnp
from jax.experimental import pallas as pl
from jax.experimental.pallas import tpu as pltpu


def _attn_kernel(xq_ref, k_ref, v_ref, ks_ref, vs_ref, mask_ref, o_ref):
    q = xq_ref[0, 0]
    k = k_ref[0, 0].astype(jnp.float32)
    scores = jax.lax.dot_general(
        q, k, (((1,), (1,)), ((), ())),
        preferred_element_type=jnp.float32)
    inv_sqrt_d = 1.0 / math.sqrt(q.shape[-1])
    scores = scores * (ks_ref[0] * inv_sqrt_d) + mask_ref[0]
    m = jnp.max(scores, axis=-1, keepdims=True)
    e = jnp.exp(scores - m)
    s = jnp.sum(e, axis=-1, keepdims=True)
    p = (e / s) * vs_ref[0]
    v = v_ref[0, 0].astype(jnp.float32)
    o_ref[0, 0] = jax.lax.dot_general(
        p, v, (((1,), (0,)), ((), ())),
        preferred_element_type=jnp.float32)


def kernel(xq, keys, values, k_scaler, v_scaler, mask):
    B, H, _, D = xq.shape
    Hkv, S = keys.shape[1], keys.shape[2]
    n_rep = H // Hkv
    xqg = xq.reshape(B, Hkv, n_rep, D)
    ks = k_scaler.reshape(B, 1, S)
    vs = v_scaler.reshape(B, 1, S)
    msk = mask.reshape(B, 1, S)
    out = pl.pallas_call(
        _attn_kernel,
        grid=(B, Hkv),
        in_specs=[
            pl.BlockSpec((1, 1, n_rep, D), lambda b, g: (b, g, 0, 0)),
            pl.BlockSpec((1, 1, S, D), lambda b, g: (b, g, 0, 0)),
            pl.BlockSpec((1, 1, S, D), lambda b, g: (b, g, 0, 0)),
            pl.BlockSpec((1, 1, S), lambda b, g: (b, 0, 0)),
            pl.BlockSpec((1, 1, S), lambda b, g: (b, 0, 0)),
            pl.BlockSpec((1, 1, S), lambda b, g: (b, 0, 0)),
        ],
        out_specs=pl.BlockSpec((1, 1, n_rep, D), lambda b, g: (b, g, 0, 0)),
        out_shape=jax.ShapeDtypeStruct((B, Hkv, n_rep, D), jnp.float32),
        compiler_params=pltpu.CompilerParams(
            dimension_semantics=("parallel", "parallel"),
        ),
        name="int8_kv_decode_attn",
    )(xqg, keys, values, ks, vs, msk)
    return out.reshape(B, H, 1, D)


# speedup vs baseline: 1.5982x; 1.5982x over previous
import math

import jax
import jax.numpy as jnp
from jax.experimental import pallas as pl
from jax.experimental.pallas import tpu as pltpu


def _attn_kernel(xq_ref, k_ref, v_ref, ks_ref, vs_ref, mask_ref, o_ref):
    G = k_ref.shape[1]
    D = xq_ref.shape[-1]
    inv_sqrt_d = 1.0 / math.sqrt(D)
    scale_row = ks_ref[0] * inv_sqrt_d
    mask_row = mask_ref[0]
    vs_row = vs_ref[0]
    for g in range(G):
        q = xq_ref[0, g]
        k = k_ref[0, g].astype(jnp.float32)
        scores = jax.lax.dot_general(
            q, k, (((1,), (1,)), ((), ())),
            preferred_element_type=jnp.float32)
        scores = scores * scale_row + mask_row
        m = jnp.max(scores, axis=-1, keepdims=True)
        e = jnp.exp(scores - m)
        s = jnp.sum(e, axis=-1, keepdims=True)
        p = e * vs_row
        v = v_ref[0, g].astype(jnp.float32)
        acc = jax.lax.dot_general(
            p, v, (((1,), (0,)), ((), ())),
            preferred_element_type=jnp.float32)
        o_ref[0, g] = acc / s


def kernel(xq, keys, values, k_scaler, v_scaler, mask):
    B, H, _, D = xq.shape
    Hkv, S = keys.shape[1], keys.shape[2]
    n_rep = H // Hkv
    xqg = xq.reshape(B, Hkv, n_rep, D)
    ks = k_scaler.reshape(B, 1, S)
    vs = v_scaler.reshape(B, 1, S)
    msk = mask.reshape(B, 1, S)
    out = pl.pallas_call(
        _attn_kernel,
        grid=(B,),
        in_specs=[
            pl.BlockSpec((1, Hkv, n_rep, D), lambda b: (b, 0, 0, 0)),
            pl.BlockSpec((1, Hkv, S, D), lambda b: (b, 0, 0, 0)),
            pl.BlockSpec((1, Hkv, S, D), lambda b: (b, 0, 0, 0)),
            pl.BlockSpec((1, 1, S), lambda b: (b, 0, 0)),
            pl.BlockSpec((1, 1, S), lambda b: (b, 0, 0)),
            pl.BlockSpec((1, 1, S), lambda b: (b, 0, 0)),
        ],
        out_specs=pl.BlockSpec((1, Hkv, n_rep, D), lambda b: (b, 0, 0, 0)),
        out_shape=jax.ShapeDtypeStruct((B, Hkv, n_rep, D), jnp.float32),
        compiler_params=pltpu.CompilerParams(
            dimension_semantics=("parallel",),
        ),
        name="int8_kv_decode_attn",
    )(xqg, keys, values, ks, vs, msk)
    return out.reshape(B, H, 1, D)
